# baseline (speedup 1.0000x reference)
import dataclasses
import functools

import jax
import jax.numpy as jnp
from jax import lax
from jax.experimental import pallas as pl
from jax.experimental.pallas import tpu as pltpu
from jax.experimental.pallas import tpu_sc as plsc

NC = 2
NS = 16
L = 16
NW = NC * NS

D = 256
W = 64
CHUNKS = 80
EPW = W * CHUNKS
E_PAD = EPW * NW
ZROWS = 10112


def _build_sc_call():
    mesh = plsc.VectorSubcoreMesh(core_axis_name="c", subcore_axis_name="s")
    cp = pltpu.CompilerParams()
    if "needs_layout_passes" in pltpu.CompilerParams.__dataclass_fields__:
        cp = dataclasses.replace(cp, needs_layout_passes=False)

    @functools.partial(
        pl.kernel,
        out_type=jax.ShapeDtypeStruct((E_PAD,), jnp.float32),
        mesh=mesh,
        scratch_types=[
            pltpu.VMEM((EPW,), jnp.int32),
            pltpu.VMEM((EPW,), jnp.int32),
            pltpu.VMEM((2 * W, D // 2), jnp.int32),
            pltpu.VMEM((2 * W, D // 2), jnp.int32),
            pltpu.VMEM((EPW,), jnp.float32),
            pltpu.VMEM((L, L), jnp.float32),
            pltpu.VMEM_SHARED((ZROWS, D // 2), jnp.int32),
            pltpu.SemaphoreType.DMA,
            pltpu.SemaphoreType.DMA,
        ],
        compiler_params=cp,
    )
    def sc_decode(z_hbm, src_hbm, dst_hbm, out_hbm, si, di, sr, dr, ob, mat,
                  zs, sem0, sem1):
        sid = lax.axis_index("s")
        wid = sid * NC + lax.axis_index("c")
        base = wid * EPW
        sems = (sem0, sem1)
        rows16 = jnp.arange(L, dtype=jnp.int32)

        nrows = ZROWS // NS
        pltpu.sync_copy(z_hbm.at[pl.ds(sid * nrows, nrows)],
                        zs.at[pl.ds(sid * nrows, nrows)])
        pltpu.sync_copy(src_hbm.at[pl.ds(base, EPW)], si)
        pltpu.sync_copy(dst_hbm.at[pl.ds(base, EPW)], di)
        plsc.subcore_barrier()

        def issue(c, b):
            dst_s = sr.at[pl.ds(b * W, W)]
            dst_d = dr.at[pl.ds(b * W, W)]
            pltpu.async_copy(zs.at[si.at[pl.ds(c * W, W)]], dst_s, sems[b])
            pltpu.async_copy(zs.at[di.at[pl.ds(c * W, W)]], dst_d, sems[b])

        def wait(b):
            pltpu.make_async_copy(
                zs.at[si.at[pl.ds(0, W)]], sr.at[pl.ds(b * W, W)], sems[b]
            ).wait()
            pltpu.make_async_copy(
                zs.at[di.at[pl.ds(0, W)]], dr.at[pl.ds(b * W, W)], sems[b]
            ).wait()

        def compute(c, b):
            @pl.loop(0, W, step=L)
            def _(g):
                mask = jnp.full((L,), -65536, jnp.int32)
                for i in range(L):
                    e = b * W + g + i
                    accs = [jnp.zeros((L,), jnp.float32) for _ in range(4)]
                    for j in range(D // (2 * L)):
                        ws = sr[e, pl.ds(j * L, L)]
                        wd = dr[e, pl.ds(j * L, L)]
                        ps = plsc.bitcast(ws, jnp.bfloat16) * plsc.bitcast(
                            wd, jnp.bfloat16)
                        pw = plsc.bitcast(ps, jnp.int32)
                        k = 2 * (j & 1)
                        accs[k] += plsc.bitcast(pw & mask, jnp.float32)
                        accs[k + 1] += plsc.bitcast(pw << 16, jnp.float32)
                    mat[i, :] = (accs[0] + accs[1]) + (accs[2] + accs[3])

                cols = [
                    plsc.load_gather(mat, [rows16, jnp.full((L,), l, jnp.int32)])
                    for l in range(L)
                ]
                while len(cols) > 1:
                    cols = [a + bcol for a, bcol in zip(cols[::2], cols[1::2])]
                tot = cols[0]
                ob[pl.ds(c * W + g, L)] = 1.0 / (1.0 + jnp.exp(-tot))

        issue(0, 0)

        @pl.loop(0, CHUNKS, step=2)
        def _(c):
            issue(c + 1, 1)
            wait(0)
            compute(c, 0)

            @pl.when(c + 2 < CHUNKS)
            def _():
                issue(c + 2, 0)

            wait(1)
            compute(c + 1, 1)

        pltpu.sync_copy(ob, out_hbm.at[pl.ds(base, EPW)])

    return sc_decode


_SC_DECODE = _build_sc_call()


def kernel(z, edge_index):
    e = edge_index.shape[1]
    src = edge_index[0].astype(jnp.int32)
    dst = edge_index[1].astype(jnp.int32)
    pad = E_PAD - e
    src = jnp.concatenate([src, jnp.zeros((pad,), jnp.int32)])
    dst = jnp.concatenate([dst, jnp.zeros((pad,), jnp.int32)])
    half = z.shape[1] // 2
    z16 = z.astype(jnp.bfloat16)
    lo = lax.bitcast_convert_type(z16[:, :half], jnp.uint16).astype(jnp.uint32)
    hi = lax.bitcast_convert_type(z16[:, half:], jnp.uint16).astype(jnp.uint32)
    z_pairs = lax.bitcast_convert_type(lo | (hi << jnp.uint32(16)), jnp.int32)
    z_pairs = jnp.pad(z_pairs, ((0, ZROWS - z_pairs.shape[0]), (0, 0)))
    out = _SC_DECODE(z_pairs, src, dst)
    return out[:e]


# speedup vs baseline: 1.1463x; 1.1463x over previous
import dataclasses
import functools

import jax
import jax.numpy as jnp
from jax import lax
from jax.experimental import pallas as pl
from jax.experimental.pallas import tpu as pltpu
from jax.experimental.pallas import tpu_sc as plsc

NC = 2
NS = 16
L = 16
NW = NC * NS

D = 256
W = 64
CHUNKS = 80
EPW = W * CHUNKS
E_PAD = EPW * NW
ZROWS = 10112


def _build_sc_call():
    mesh = plsc.VectorSubcoreMesh(core_axis_name="c", subcore_axis_name="s")
    cp = pltpu.CompilerParams()
    if "needs_layout_passes" in pltpu.CompilerParams.__dataclass_fields__:
        cp = dataclasses.replace(cp, needs_layout_passes=False)

    @functools.partial(
        pl.kernel,
        out_type=jax.ShapeDtypeStruct((E_PAD,), jnp.float32),
        mesh=mesh,
        scratch_types=[
            pltpu.VMEM((EPW,), jnp.int32),
            pltpu.VMEM((EPW,), jnp.int32),
            pltpu.VMEM((2 * W, D // 2), jnp.int32),
            pltpu.VMEM((2 * W, D // 2), jnp.int32),
            pltpu.VMEM((EPW,), jnp.float32),
            pltpu.VMEM((L, L), jnp.float32),
            pltpu.VMEM_SHARED((ZROWS, D // 2), jnp.int32),
            pltpu.SemaphoreType.DMA,
            pltpu.SemaphoreType.DMA,
        ],
        compiler_params=cp,
    )
    def sc_decode(z_hbm, src_hbm, dst_hbm, out_hbm, si, di, sr, dr, ob, mat,
                  zs, sem0, sem1):
        sid = lax.axis_index("s")
        wid = sid * NC + lax.axis_index("c")
        base = wid * EPW
        sems = (sem0, sem1)
        rows16 = jnp.arange(L, dtype=jnp.int32)

        nrows = ZROWS // NS
        pltpu.sync_copy(z_hbm.at[pl.ds(sid * nrows, nrows)],
                        zs.at[pl.ds(sid * nrows, nrows)])
        pltpu.sync_copy(src_hbm.at[pl.ds(base, EPW)], si)
        pltpu.sync_copy(dst_hbm.at[pl.ds(base, EPW)], di)
        plsc.subcore_barrier()

        def issue(c, b):
            dst_s = sr.at[pl.ds(b * W, W)]
            dst_d = dr.at[pl.ds(b * W, W)]
            pltpu.async_copy(zs.at[si.at[pl.ds(c * W, W)]], dst_s, sems[b])
            pltpu.async_copy(zs.at[di.at[pl.ds(c * W, W)]], dst_d, sems[b])

        def wait(b):
            pltpu.make_async_copy(
                zs.at[si.at[pl.ds(0, W)]], sr.at[pl.ds(b * W, W)], sems[b]
            ).wait()
            pltpu.make_async_copy(
                zs.at[di.at[pl.ds(0, W)]], dr.at[pl.ds(b * W, W)], sems[b]
            ).wait()

        def compute(c, b):
            @pl.loop(0, W, step=L)
            def _(g):
                mask = jnp.full((L,), -65536, jnp.int32)
                for i in range(L):
                    e = b * W + g + i
                    acc0 = jnp.zeros((L,), jnp.float32)
                    acc1 = jnp.zeros((L,), jnp.float32)
                    for j in range(D // (2 * L)):
                        ws = sr[e, pl.ds(j * L, L)]
                        wd = dr[e, pl.ds(j * L, L)]
                        ps = plsc.bitcast(ws, jnp.bfloat16) * plsc.bitcast(
                            wd, jnp.bfloat16)
                        pw = plsc.bitcast(ps, jnp.int32)
                        acc0 += plsc.bitcast(pw & mask, jnp.float32)
                        acc1 += plsc.bitcast(pw << 16, jnp.float32)
                    mat[i, :] = acc0 + acc1

                tot = plsc.load_gather(mat, [rows16, jnp.zeros((L,), jnp.int32)])
                for l in range(1, L):
                    tot += plsc.load_gather(
                        mat, [rows16, jnp.full((L,), l, jnp.int32)]
                    )
                ob[pl.ds(c * W + g, L)] = 1.0 / (1.0 + jnp.exp(-tot))

        issue(0, 0)

        @pl.loop(0, CHUNKS, step=2)
        def _(c):
            issue(c + 1, 1)
            wait(0)
            compute(c, 0)

            @pl.when(c + 2 < CHUNKS)
            def _():
                issue(c + 2, 0)

            wait(1)
            compute(c + 1, 1)

        pltpu.sync_copy(ob, out_hbm.at[pl.ds(base, EPW)])

    return sc_decode


_SC_DECODE = _build_sc_call()


def kernel(z, edge_index):
    e = edge_index.shape[1]
    src = edge_index[0].astype(jnp.int32)
    dst = edge_index[1].astype(jnp.int32)
    pad = E_PAD - e
    src = jnp.concatenate([src, jnp.zeros((pad,), jnp.int32)])
    dst = jnp.concatenate([dst, jnp.zeros((pad,), jnp.int32)])
    half = z.shape[1] // 2
    z16 = z.astype(jnp.bfloat16)
    lo = lax.bitcast_convert_type(z16[:, :half], jnp.uint16).astype(jnp.uint32)
    hi = lax.bitcast_convert_type(z16[:, half:], jnp.uint16).astype(jnp.uint32)
    z_pairs = lax.bitcast_convert_type(lo | (hi << jnp.uint32(16)), jnp.int32)
    z_pairs = jnp.pad(z_pairs, ((0, ZROWS - z_pairs.shape[0]), (0, 0)))
    out = _SC_DECODE(z_pairs, src, dst)
    return out[:e]


# speedup vs baseline: 1.4227x; 1.2411x over previous
import dataclasses
import functools

import jax
import jax.numpy as jnp
from jax import lax
from jax.experimental import pallas as pl
from jax.experimental.pallas import tpu as pltpu
from jax.experimental.pallas import tpu_sc as plsc

NC = 2
NS = 16
L = 16
NW = NC * NS

D = 256
W = 32
CHUNKS = 160
EPW = W * CHUNKS
E_PAD = EPW * NW
ZROWS = 10112


def _build_sc_call():
    mesh = plsc.VectorSubcoreMesh(core_axis_name="c", subcore_axis_name="s")
    cp = pltpu.CompilerParams()
    if "needs_layout_passes" in pltpu.CompilerParams.__dataclass_fields__:
        cp = dataclasses.replace(cp, needs_layout_passes=False)

    @functools.partial(
        pl.kernel,
        out_type=jax.ShapeDtypeStruct((E_PAD,), jnp.float32),
        mesh=mesh,
        scratch_types=[
            pltpu.VMEM((EPW,), jnp.int32),
            pltpu.VMEM((EPW,), jnp.int32),
            pltpu.VMEM((2 * W, D // 2), jnp.int32),
            pltpu.VMEM((2 * W, D // 2), jnp.int32),
            pltpu.VMEM((EPW,), jnp.float32),
            pltpu.VMEM((L, L), jnp.float32),
            pltpu.VMEM((L, L), jnp.float32),
            pltpu.VMEM_SHARED((ZROWS, D // 2), jnp.int32),
            pltpu.SemaphoreType.DMA,
            pltpu.SemaphoreType.DMA,
        ],
        compiler_params=cp,
    )
    def sc_decode(z_hbm, src_hbm, dst_hbm, out_hbm, si, di, sr, dr, ob, mat,
                  mat2, zs, sem0, sem1):
        sid = lax.axis_index("s")
        wid = sid * NC + lax.axis_index("c")
        base = wid * EPW
        sems = (sem0, sem1)
        rows16 = jnp.arange(L, dtype=jnp.int32)

        nrows = ZROWS // NS
        pltpu.sync_copy(z_hbm.at[pl.ds(sid * nrows, nrows)],
                        zs.at[pl.ds(sid * nrows, nrows)])
        pltpu.sync_copy(src_hbm.at[pl.ds(base, EPW)], si)
        pltpu.sync_copy(dst_hbm.at[pl.ds(base, EPW)], di)
        plsc.subcore_barrier()

        def issue(c, b):
            dst_s = sr.at[pl.ds(b * W, W)]
            dst_d = dr.at[pl.ds(b * W, W)]
            pltpu.async_copy(zs.at[si.at[pl.ds(c * W, W)]], dst_s, sems[b])
            pltpu.async_copy(zs.at[di.at[pl.ds(c * W, W)]], dst_d, sems[b])

        def wait(b):
            pltpu.make_async_copy(
                zs.at[si.at[pl.ds(0, W)]], sr.at[pl.ds(b * W, W)], sems[b]
            ).wait()
            pltpu.make_async_copy(
                zs.at[di.at[pl.ds(0, W)]], dr.at[pl.ds(b * W, W)], sems[b]
            ).wait()

        mask = jnp.full((L,), -65536, jnp.int32)

        def macs(b, g, m):
            for i in range(L):
                e = b * W + g + i
                acc0 = jnp.zeros((L,), jnp.float32)
                acc1 = jnp.zeros((L,), jnp.float32)
                for j in range(D // (2 * L)):
                    ws = sr[e, pl.ds(j * L, L)]
                    wd = dr[e, pl.ds(j * L, L)]
                    ps = plsc.bitcast(ws, jnp.bfloat16) * plsc.bitcast(
                        wd, jnp.bfloat16)
                    pw = plsc.bitcast(ps, jnp.int32)
                    acc0 += plsc.bitcast(pw & mask, jnp.float32)
                    acc1 += plsc.bitcast(pw << 16, jnp.float32)
                m[i, :] = acc0 + acc1

        def trans(c, g, m):
            tot = plsc.load_gather(m, [rows16, jnp.zeros((L,), jnp.int32)])
            for l in range(1, L):
                tot += plsc.load_gather(
                    m, [rows16, jnp.full((L,), l, jnp.int32)]
                )
            ob[pl.ds(c * W + g, L)] = 1.0 / (1.0 + jnp.exp(-tot))

        def compute(c, b):
            @pl.loop(0, W, step=2 * L)
            def _(g):
                macs(b, g, mat)
                macs(b, g + L, mat2)
                trans(c, g, mat)
                trans(c, g + L, mat2)

        issue(0, 0)

        @pl.loop(0, CHUNKS, step=2)
        def _(c):
            issue(c + 1, 1)
            wait(0)
            compute(c, 0)

            @pl.when(c + 2 < CHUNKS)
            def _():
                issue(c + 2, 0)

            wait(1)
            compute(c + 1, 1)

        pltpu.sync_copy(ob, out_hbm.at[pl.ds(base, EPW)])

    return sc_decode


_SC_DECODE = _build_sc_call()


def kernel(z, edge_index):
    e = edge_index.shape[1]
    src = edge_index[0].astype(jnp.int32)
    dst = edge_index[1].astype(jnp.int32)
    pad = E_PAD - e
    src = jnp.concatenate([src, jnp.zeros((pad,), jnp.int32)])
    dst = jnp.concatenate([dst, jnp.zeros((pad,), jnp.int32)])
    half = z.shape[1] // 2
    z16 = z.astype(jnp.bfloat16)
    lo = lax.bitcast_convert_type(z16[:, :half], jnp.uint16).astype(jnp.uint32)
    hi = lax.bitcast_convert_type(z16[:, half:], jnp.uint16).astype(jnp.uint32)
    z_pairs = lax.bitcast_convert_type(lo | (hi << jnp.uint32(16)), jnp.int32)
    z_pairs = jnp.pad(z_pairs, ((0, ZROWS - z_pairs.shape[0]), (0, 0)))
    out = _SC_DECODE(z_pairs, src, dst)
    return out[:e]
